# baseline (speedup 1.0000x reference)
import functools
import math

import jax
import jax.numpy as jnp
from jax import lax
from jax.experimental import pallas as pl
from jax.experimental.pallas import tpu as pltpu
from jax.experimental.pallas import tpu_sc as plsc

_SCALE = 8.0


@functools.lru_cache(maxsize=None)
def _make(V, D, B):
    info = plsc.get_sparse_core_info()
    NC, NS, L = info.num_cores, info.num_subcores, info.num_lanes
    NW = NC * NS
    assert B % NW == 0
    b_per_w = B // NW
    C = 128
    assert b_per_w % C == 0
    n_chunks = b_per_w // C
    mesh = plsc.VectorSubcoreMesh(core_axis_name="c", subcore_axis_name="s")

    @functools.partial(
        pl.kernel,
        mesh=mesh,
        compiler_params=pltpu.CompilerParams(use_tc_tiling_on_sc=False),
        out_type=jax.ShapeDtypeStruct((B, D), jnp.float32),
        scratch_types=[
            pltpu.VMEM((b_per_w,), jnp.int32),
            pltpu.VMEM((C, D), jnp.float32),
            pltpu.SemaphoreType.DMA,
        ],
    )
    def k(table_hbm, idx_hbm, out_hbm, idx_v, rows_v, sem):
        wid = lax.axis_index("s") * NC + lax.axis_index("c")
        base = wid * b_per_w
        pltpu.sync_copy(idx_hbm.at[pl.ds(base, b_per_w)], idx_v)

        def chunk(j, carry):
            off = pl.multiple_of(j * C, C)
            pltpu.async_copy(
                table_hbm.at[idx_v.at[pl.ds(off, C)]], rows_v, sem
            ).wait()

            def srow(r, c2):
                for c in range(D // L):
                    rows_v[r, pl.ds(c * L, L)] = (
                        rows_v[r, pl.ds(c * L, L)] * _SCALE
                    )
                return c2

            lax.fori_loop(0, C, srow, 0, unroll=2)
            pltpu.sync_copy(rows_v, out_hbm.at[pl.ds(base + off, C)])
            return carry

        lax.fori_loop(0, n_chunks, chunk, 0)

    return k


def kernel(tokens, table):
    Bt, S = tokens.shape
    V, D = table.shape
    B = Bt * S
    idx = tokens.reshape(B).astype(jnp.int32)
    out = _make(V, D, B)(table, idx)
    return out.reshape(Bt, S, D)


# speedup vs baseline: 1.1602x; 1.1602x over previous
import functools

import jax
import jax.numpy as jnp
from jax import lax
from jax.experimental import pallas as pl
from jax.experimental.pallas import tpu as pltpu
from jax.experimental.pallas import tpu_sc as plsc

_SCALE = 8.0


@functools.lru_cache(maxsize=None)
def _make(V, D, B):
    info = plsc.get_sparse_core_info()
    NC, NS, L = info.num_cores, info.num_subcores, info.num_lanes
    NW = NC * NS
    assert B % NW == 0
    b_per_w = B // NW
    C = 128
    NBUF = 8
    assert b_per_w % (C * NBUF) == 0
    n_chunks = b_per_w // C
    n_groups = n_chunks // NBUF
    mesh = plsc.VectorSubcoreMesh(core_axis_name="c", subcore_axis_name="s")

    @functools.partial(
        pl.kernel,
        mesh=mesh,
        compiler_params=pltpu.CompilerParams(use_tc_tiling_on_sc=False),
        out_type=jax.ShapeDtypeStruct((B, D), jnp.float32),
        scratch_types=[
            pltpu.VMEM((b_per_w,), jnp.int32),
            pltpu.VMEM((NBUF, C, D), jnp.float32),
            [pltpu.SemaphoreType.DMA] * NBUF,
            [pltpu.SemaphoreType.DMA] * NBUF,
        ],
    )
    def k(table_hbm, idx_hbm, out_hbm, idx_v, rows_v, gsems, ssems):
        wid = lax.axis_index("s") * NC + lax.axis_index("c")
        base = wid * b_per_w
        pltpu.sync_copy(idx_hbm.at[pl.ds(base, b_per_w)], idx_v)

        def fire_gather(j, b):
            off = pl.multiple_of(j * C, C)
            pltpu.async_copy(
                table_hbm.at[idx_v.at[pl.ds(off, C)]],
                rows_v.at[b],
                gsems[b],
            )

        for b in range(NBUF):
            fire_gather(b, b)

        def group(g, carry):
            for b in range(NBUF):
                j = g * NBUF + b
                off = pl.multiple_of(j * C, C)
                pltpu.make_async_copy(
                    table_hbm.at[idx_v.at[pl.ds(off, C)]],
                    rows_v.at[b],
                    gsems[b],
                ).wait()

                @plsc.parallel_loop(0, C, unroll=4)
                def srow(r):
                    for c in range(D // L):
                        rows_v[b, r, pl.ds(c * L, L)] = (
                            rows_v[b, r, pl.ds(c * L, L)] * _SCALE
                        )

                pltpu.async_copy(
                    rows_v.at[b], out_hbm.at[pl.ds(base + off, C)], ssems[b]
                )

                nxt = j + NBUF

                @pl.when(nxt < n_chunks)
                def _():
                    pltpu.make_async_copy(
                        rows_v.at[b],
                        out_hbm.at[pl.ds(base + off, C)],
                        ssems[b],
                    ).wait()
                    fire_gather(nxt, b)

            return carry

        lax.fori_loop(0, n_groups, group, 0)

        last = (n_groups - 1) * NBUF
        for b in range(NBUF):
            off = pl.multiple_of((last + b) * C, C)
            pltpu.make_async_copy(
                rows_v.at[b], out_hbm.at[pl.ds(base + off, C)], ssems[b]
            ).wait()

    return k


def kernel(tokens, table):
    Bt, S = tokens.shape
    V, D = table.shape
    B = Bt * S
    idx = tokens.reshape(B).astype(jnp.int32)
    out = _make(V, D, B)(table, idx)
    return out.reshape(Bt, S, D)
